# baseline (speedup 1.0000x reference)
import functools

import jax
import jax.numpy as jnp
from jax import lax
from jax.experimental import pallas as pl
from jax.experimental.pallas import tpu as pltpu
from jax.experimental.pallas import tpu_sc as plsc

BATCH = 16384
EMBED_K = 16
NUM_CORES = 2
NUM_SUBCORES = 16
NUM_WORKERS = NUM_CORES * NUM_SUBCORES
BPW = BATCH // NUM_WORKERS
NCHUNK = 4
CHUNK = BPW // NCHUNK
NBLK = BPW // 16


def _build():
    mesh = plsc.VectorSubcoreMesh(core_axis_name="c", subcore_axis_name="s")

    @functools.partial(
        pl.kernel,
        mesh=mesh,
        compiler_params=pltpu.CompilerParams(
            needs_layout_passes=False, use_tc_tiling_on_sc=False),
        out_type=jax.ShapeDtypeStruct((BATCH,), jnp.float32),
        scratch_types=[
            pltpu.VMEM((NCHUNK, CHUNK), jnp.int32),
            pltpu.VMEM((NCHUNK, CHUNK), jnp.int32),
            pltpu.VMEM((BPW, EMBED_K), jnp.float32),
            pltpu.VMEM((BPW, EMBED_K), jnp.float32),
            pltpu.VMEM((BPW,), jnp.float32),
            pltpu.VMEM((16 * EMBED_K,), jnp.float32),
            pltpu.SemaphoreType.DMA,
        ],
    )
    def body(xu_hbm, xi_hbm, w_hbm, h_hbm, out_hbm, iu, ii, u, v, o, p1, sem):
        wid = lax.axis_index("s") * NUM_CORES + lax.axis_index("c")

        pltpu.sync_copy(xu_hbm.at[wid], iu)
        pltpu.sync_copy(xi_hbm.at[wid], ii)

        copies = []
        for j in range(NCHUNK):
            dst = pl.ds(j * CHUNK, CHUNK)
            copies.append(pltpu.async_copy(w_hbm.at[iu.at[j]], u.at[dst], sem))
            copies.append(pltpu.async_copy(h_hbm.at[ii.at[j]], v.at[dst], sem))
        for c in copies:
            c.wait()

        lane = lax.iota(jnp.int32, 16)
        cols = [lane * EMBED_K + k for k in range(EMBED_K)]

        def blk(b, carry):
            base = b * 16
            for r in range(16):
                p1[pl.ds(r * EMBED_K, EMBED_K)] = u[base + r] * v[base + r]
            acc = plsc.load_gather(p1, [cols[0]])
            for k in range(1, EMBED_K):
                acc = acc + plsc.load_gather(p1, [cols[k]])
            o[pl.ds(base, 16)] = 1.0 / (1.0 + jnp.exp(-acc))
            return carry

        lax.fori_loop(0, NBLK, blk, 0)

        pltpu.sync_copy(o, out_hbm.at[pl.ds(wid * BPW, BPW)])

    return body


_KERNEL = _build()


def kernel(x, W, H):
    xu = x[:, 0].reshape(NUM_WORKERS, NCHUNK, CHUNK)
    xi = x[:, 1].reshape(NUM_WORKERS, NCHUNK, CHUNK)
    return _KERNEL(xu, xi, W, H)
